# baseline (speedup 1.0000x reference)
import functools

import jax
import jax.numpy as jnp
from jax.experimental import pallas as pl
from jax.experimental.pallas import tpu as pltpu
from jax.experimental.pallas import tpu_sc as plsc

B = 16384
D = 64
NC = 2
NS = 16
NW = NC * NS
RPW = B // NW
CHUNK = 128
NCHUNK = RPW // CHUNK


def _sc_gather_body(users, movies, lmf_uw, lmf_mw, mlp_uw, mlp_mw,
                    lmf_ub, lmf_mb,
                    uw_out, mw_out, mlp_u_out, mlp_m_out, ub_out, mb_out,
                    idx_u, idx_m, buf_a, buf_b, bias_u, bias_m, sem_a, sem_b):
    wid = jax.lax.axis_index("s") * NC + jax.lax.axis_index("c")
    base = wid * RPW

    pltpu.sync_copy(users.at[pl.ds(wid * NCHUNK, NCHUNK)], idx_u)
    pltpu.sync_copy(movies.at[pl.ds(wid * NCHUNK, NCHUNK)], idx_m)

    def fire(table, idx, buf, sem):
        return [
            pltpu.async_copy(table.at[idx.at[j]],
                             buf.at[pl.ds(j * CHUNK, CHUNK)], sem)
            for j in range(NCHUNK)
        ]

    cps_a = fire(lmf_uw, idx_u, buf_a, sem_a)
    cps_b = fire(mlp_uw, idx_u, buf_b, sem_b)
    for c in cps_a:
        c.wait()
    pltpu.sync_copy(buf_a, uw_out.at[pl.ds(base, RPW)])
    cps_a = fire(lmf_mw, idx_m, buf_a, sem_a)
    for c in cps_b:
        c.wait()
    pltpu.sync_copy(buf_b, mlp_u_out.at[pl.ds(base, RPW)])
    cps_b = fire(mlp_mw, idx_m, buf_b, sem_b)
    for c in cps_a:
        c.wait()
    pltpu.sync_copy(buf_a, mw_out.at[pl.ds(base, RPW)])
    cps_bias = [
        pltpu.async_copy(lmf_ub.at[idx_u.at[j]],
                         bias_u.at[pl.ds(j * CHUNK, CHUNK)], sem_a)
        for j in range(NCHUNK)
    ] + [
        pltpu.async_copy(lmf_mb.at[idx_m.at[j]],
                         bias_m.at[pl.ds(j * CHUNK, CHUNK)], sem_a)
        for j in range(NCHUNK)
    ]
    for c in cps_b:
        c.wait()
    pltpu.sync_copy(buf_b, mlp_m_out.at[pl.ds(base, RPW)])
    for c in cps_bias:
        c.wait()
    pltpu.sync_copy(bias_u, ub_out.at[pl.ds(base, RPW)])
    pltpu.sync_copy(bias_m, mb_out.at[pl.ds(base, RPW)])


_sc_gather = functools.partial(
    pl.kernel,
    out_type=[
        jax.ShapeDtypeStruct((B, D), jnp.float32),
        jax.ShapeDtypeStruct((B, D), jnp.float32),
        jax.ShapeDtypeStruct((B, D), jnp.float32),
        jax.ShapeDtypeStruct((B, D), jnp.float32),
        jax.ShapeDtypeStruct((B,), jnp.float32),
        jax.ShapeDtypeStruct((B,), jnp.float32),
    ],
    mesh=plsc.VectorSubcoreMesh(
        core_axis_name="c", subcore_axis_name="s", num_cores=NC,
        num_subcores=NS),
    scratch_types=[
        pltpu.VMEM((NCHUNK, CHUNK), jnp.int32),
        pltpu.VMEM((NCHUNK, CHUNK), jnp.int32),
        pltpu.VMEM((RPW, D), jnp.float32),
        pltpu.VMEM((RPW, D), jnp.float32),
        pltpu.VMEM((RPW,), jnp.float32),
        pltpu.VMEM((RPW,), jnp.float32),
        pltpu.SemaphoreType.DMA,
        pltpu.SemaphoreType.DMA,
    ],
    compiler_params=pltpu.CompilerParams(use_tc_tiling_on_sc=False),
)(_sc_gather_body)


RB = 2048


def _tc_dense_body(uw, mw, mlp_u, mlp_m, ub, mb,
                   W1, b1, W2, b2, W3, b3, Wf, bf, out):
    lmf = jnp.sum(uw[...] * mw[...], axis=1, keepdims=True) + ub[...] + mb[...]
    lmf = jax.nn.sigmoid(lmf)
    W1a = W1[0:D, :]
    W1b = W1[D:2 * D, :]
    h = jnp.dot(mlp_u[...], W1a, preferred_element_type=jnp.float32)
    h += jnp.dot(mlp_m[...], W1b, preferred_element_type=jnp.float32)
    h = jax.nn.relu(h + b1[...])
    h = jax.nn.relu(jnp.dot(h, W2[...], preferred_element_type=jnp.float32)
                    + b2[...])
    mlp = jax.nn.sigmoid(
        jnp.dot(h, W3[...], preferred_element_type=jnp.float32) + b3[...])
    x = jax.nn.sigmoid(lmf * Wf[0, 0] + mlp * Wf[1, 0] + bf[0, 0])
    out[...] = x * 4.5 + 0.5


def _tc_dense(uw, mw, mlp_u, mlp_m, ub, mb, W1, b1, W2, b2, W3, b3, Wf, bf):
    grid = (B // RB,)
    row = lambda i: (i, 0)
    rep = lambda i: (0, 0)
    return pl.pallas_call(
        _tc_dense_body,
        grid=grid,
        in_specs=[
            pl.BlockSpec((RB, D), row),
            pl.BlockSpec((RB, D), row),
            pl.BlockSpec((RB, D), row),
            pl.BlockSpec((RB, D), row),
            pl.BlockSpec((RB, 1), row),
            pl.BlockSpec((RB, 1), row),
            pl.BlockSpec((2 * D, D), rep),
            pl.BlockSpec((1, D), rep),
            pl.BlockSpec((D, 16), rep),
            pl.BlockSpec((1, 16), rep),
            pl.BlockSpec((16, 1), rep),
            pl.BlockSpec((1, 1), rep),
            pl.BlockSpec((2, 1), rep),
            pl.BlockSpec((1, 1), rep),
        ],
        out_specs=pl.BlockSpec((RB, 1), row),
        out_shape=jax.ShapeDtypeStruct((B, 1), jnp.float32),
    )(uw, mw, mlp_u, mlp_m, ub, mb, W1, b1, W2, b2, W3, b3, Wf, bf)


def kernel(users, movies, lmf_user_w, lmf_user_b, lmf_movie_w, lmf_movie_b,
           mlp_user_w, mlp_movie_w, W1, b1, W2, b2, W3, b3, Wf, bf):
    users2d = users.astype(jnp.int32).reshape(B // CHUNK, CHUNK)
    movies2d = movies.astype(jnp.int32).reshape(B // CHUNK, CHUNK)
    uw, mw, mlp_u, mlp_m, ub, mb = _sc_gather(
        users2d, movies2d, lmf_user_w, lmf_movie_w, mlp_user_w, mlp_movie_w,
        lmf_user_b.reshape(-1), lmf_movie_b.reshape(-1))
    return _tc_dense(
        uw, mw, mlp_u, mlp_m, ub.reshape(B, 1), mb.reshape(B, 1),
        W1, b1.reshape(1, D), W2, b2.reshape(1, 16), W3, b3.reshape(1, 1),
        Wf, bf.reshape(1, 1))
